# baseline (speedup 1.0000x reference)
import functools

import jax
import jax.numpy as jnp
from jax import lax
from jax.experimental import pallas as pl
from jax.experimental.pallas import tpu as pltpu
from jax.experimental.pallas import tpu_sc as plsc

_NC = 2
_NS = 16
_L = 16
_NW = _NC * _NS
_CH = 128


def _sc_call(B, D, BPW, NCH):
    mesh = plsc.VectorSubcoreMesh(
        core_axis_name="c", subcore_axis_name="s",
        num_cores=_NC, num_subcores=_NS,
    )

    @functools.partial(
        pl.kernel,
        mesh=mesh,
        out_type=[
            jax.ShapeDtypeStruct((B,), jnp.float32),
            jax.ShapeDtypeStruct((B,), jnp.float32),
            jax.ShapeDtypeStruct((D, B), jnp.float32),
            jax.ShapeDtypeStruct((D, B), jnp.float32),
        ],
        scratch_types=[
            pltpu.VMEM((BPW,), jnp.int32),
            pltpu.VMEM((BPW,), jnp.int32),
            pltpu.VMEM((D, BPW), jnp.float32),
            pltpu.VMEM((D, BPW), jnp.float32),
            pltpu.VMEM((BPW,), jnp.float32),
            pltpu.VMEM((BPW,), jnp.float32),
            pltpu.SemaphoreType.DMA,
        ],
        compiler_params=pltpu.CompilerParams(
            needs_layout_passes=False, use_tc_tiling_on_sc=False,
        ),
    )
    def run(user_h, item_h, bi_h, gut_h, git_h,
            xui_o, beta_o, gut_o, git_o,
            idx_u, idx_i, gu_v, gi_v, beta_v, xui_v, sem):
        wid = lax.axis_index("s") * _NC + lax.axis_index("c")
        base = wid * BPW

        pltpu.sync_copy(user_h.at[pl.ds(base, BPW)], idx_u)
        pltpu.sync_copy(item_h.at[pl.ds(base, BPW)], idx_i)

        copies = []
        for c in range(NCH):
            sl = pl.ds(c * _CH, _CH)
            iu = idx_u.at[sl]
            ii = idx_i.at[sl]
            copies.append(pltpu.async_copy(bi_h.at[ii], beta_v.at[sl], sem))
            for j in range(D):
                copies.append(pltpu.async_copy(
                    gut_h.at[j].at[iu], gu_v.at[j].at[sl], sem))
                copies.append(pltpu.async_copy(
                    git_h.at[j].at[ii], gi_v.at[j].at[sl], sem))
        for cp in copies:
            cp.wait()

        def body(t, carry):
            sl = pl.ds(t * _L, _L)
            acc = beta_v[sl]
            for j in range(D):
                acc = acc + gu_v[j, sl] * gi_v[j, sl]
            xui_v[sl] = acc
            return carry

        lax.fori_loop(0, BPW // _L, body, 0)

        out_sl = pl.ds(base, BPW)
        pltpu.sync_copy(xui_v, xui_o.at[out_sl])
        pltpu.sync_copy(beta_v, beta_o.at[out_sl])
        pltpu.sync_copy(gu_v, gut_o.at[:, out_sl])
        pltpu.sync_copy(gi_v, git_o.at[:, out_sl])

    return run


def kernel(user, item, Bi, Gu, Gi):
    B = user.shape[0]
    D = Gu.shape[1]
    BPW = B // _NW
    NCH = BPW // _CH

    run = _sc_call(B, D, BPW, NCH)
    xui, beta, gut_g, git_g = run(
        user.astype(jnp.int32), item.astype(jnp.int32), Bi, Gu.T, Gi.T)
    return (xui, beta, gut_g.T, git_g.T)


# speedup vs baseline: 3.0844x; 3.0844x over previous
import functools

import jax
import jax.numpy as jnp
from jax import lax
from jax.experimental import pallas as pl
from jax.experimental.pallas import tpu as pltpu
from jax.experimental.pallas import tpu_sc as plsc

_NC = 2
_NS = 16
_L = 16
_NW = _NC * _NS
_CH = 128


def _sc_call(B, D, BPW, NCH):
    mesh = plsc.VectorSubcoreMesh(
        core_axis_name="c", subcore_axis_name="s",
        num_cores=_NC, num_subcores=_NS,
    )

    @functools.partial(
        pl.kernel,
        mesh=mesh,
        out_type=[
            jax.ShapeDtypeStruct((B,), jnp.float32),
            jax.ShapeDtypeStruct((B,), jnp.float32),
            jax.ShapeDtypeStruct((B, D), jnp.float32),
            jax.ShapeDtypeStruct((B, D), jnp.float32),
        ],
        scratch_types=[
            pltpu.VMEM((NCH, _CH), jnp.int32),
            pltpu.VMEM((NCH, _CH), jnp.int32),
            pltpu.VMEM((BPW, D), jnp.float32),
            pltpu.VMEM((BPW, D), jnp.float32),
            pltpu.VMEM((BPW,), jnp.float32),
            pltpu.VMEM((BPW,), jnp.float32),
            pltpu.SemaphoreType.DMA,
        ],
        compiler_params=pltpu.CompilerParams(
            needs_layout_passes=False, use_tc_tiling_on_sc=False,
        ),
    )
    def run(user_h, item_h, bi_h, gu_h, gi_h,
            xui_o, beta_o, gu_o, gi_o,
            idx_u, idx_i, gu_v, gi_v, beta_v, xui_v, sem):
        wid = lax.axis_index("s") * _NC + lax.axis_index("c")
        base = wid * BPW

        pltpu.sync_copy(user_h.at[wid], idx_u)
        pltpu.sync_copy(item_h.at[wid], idx_i)

        copies = []
        for c in range(NCH):
            sl = pl.ds(c * _CH, _CH)
            copies.append(pltpu.async_copy(gu_h.at[idx_u.at[c]], gu_v.at[sl], sem))
            copies.append(pltpu.async_copy(gi_h.at[idx_i.at[c]], gi_v.at[sl], sem))
            copies.append(pltpu.async_copy(bi_h.at[idx_i.at[c]], beta_v.at[sl], sem))
        for cp in copies:
            cp.wait()

        iota = lax.iota(jnp.int32, _L)

        def body(t, carry):
            rows = t * _L + iota
            acc = beta_v[pl.ds(t * _L, _L)]
            for j in range(D):
                colj = jnp.full((_L,), j, jnp.int32)
                acc = acc + (plsc.load_gather(gu_v, [rows, colj])
                             * plsc.load_gather(gi_v, [rows, colj]))
            xui_v[pl.ds(t * _L, _L)] = acc
            return carry

        lax.fori_loop(0, BPW // _L, body, 0)

        out_sl = pl.ds(base, BPW)
        pltpu.sync_copy(xui_v, xui_o.at[out_sl])
        pltpu.sync_copy(beta_v, beta_o.at[out_sl])
        pltpu.sync_copy(gu_v, gu_o.at[out_sl])
        pltpu.sync_copy(gi_v, gi_o.at[out_sl])

    return run


def kernel(user, item, Bi, Gu, Gi):
    B = user.shape[0]
    D = Gu.shape[1]
    BPW = B // _NW
    NCH = BPW // _CH

    user_r = user.astype(jnp.int32).reshape(_NW, NCH, _CH)
    item_r = item.astype(jnp.int32).reshape(_NW, NCH, _CH)

    run = _sc_call(B, D, BPW, NCH)
    xui, beta, gu_g, gi_g = run(user_r, item_r, Bi, Gu, Gi)
    return (xui, beta, gu_g, gi_g)
